# baseline (speedup 1.0000x reference)
import numpy as np
import jax
import jax.numpy as jnp
from jax.experimental import pallas as pl
from jax.experimental.pallas import tpu as pltpu

_HI = jax.lax.Precision.HIGHEST
_NB = 4


def _make_body(nb, c, hh, ww):
    inv_hw = np.float32(1.0 / (hh * ww))

    def body(x_ref, ya_ref, yb_ref, wf_ref, w1_ref, w2_ref, o_ref):
        x = x_ref[...]

        def taps(y_ref):
            y7 = y_ref[...].reshape(nb, hh // 2, 2, ww, 2, c // 128, 128)
            return (y7[:, :, 0, :, 0] + y7[:, :, 0, :, 1] +
                    y7[:, :, 1, :, 0] + y7[:, :, 1, :, 1])

        yp = jnp.concatenate([taps(ya_ref), taps(yb_ref)], axis=1) \
            .reshape(nb, hh * ww, c)

        dn = (((2,), (1,)), ((0,), (0,)))
        ones = jnp.full((nb, 1, hh * ww), inv_hw, jnp.float32)
        ones4 = jnp.full((nb, 1, hh * ww), inv_hw * np.float32(0.25),
                         jnp.float32)
        xg = jax.lax.dot_general(ones, x, dn,
                                 preferred_element_type=jnp.float32)
        yg = jax.lax.dot_general(ones4, yp, dn,
                                 preferred_element_type=jnp.float32)
        feat = jnp.concatenate([xg, yg], axis=2)[:, 0, :]

        common = jnp.maximum(
            jax.lax.dot_general(feat, wf_ref[...],
                                (((1,), (1,)), ((), ())), precision=_HI,
                                preferred_element_type=jnp.float32),
            0.0)
        xw = jax.nn.sigmoid(
            jnp.dot(common, w1_ref[...], precision=_HI,
                    preferred_element_type=jnp.float32))
        yw = jax.nn.sigmoid(
            jnp.dot(common, w2_ref[...], precision=_HI,
                    preferred_element_type=jnp.float32)) \
            * np.float32(0.25)

        o_ref[...] = x * xw[:, None, :] + yw[:, None, :] * yp

    return body


@jax.jit
def kernel(x, y, w_fc, w_fc1, w_fc2):
    B, C, H, W = x.shape
    assert y.shape == (B, C, 2 * H, 2 * W)
    hidden = w_fc.shape[1]

    nb = _NB if B % _NB == 0 else 1
    xt = jax.lax.transpose(x.astype(jnp.float32), (0, 2, 3, 1))
    yt = jax.lax.transpose(y.astype(jnp.float32), (0, 2, 3, 1))
    xr = xt.reshape(B, H * W, C)
    yr = yt.reshape(B, 4 * H * W, C)

    out = pl.pallas_call(
        _make_body(nb, C, H, W),
        grid=(B // nb,),
        in_specs=[
            pl.BlockSpec((nb, H * W, C), lambda b: (b, 0, 0)),
            pl.BlockSpec((nb, 2 * H * W, C), lambda b: (b, 0, 0)),
            pl.BlockSpec((nb, 2 * H * W, C), lambda b: (b, 1, 0)),
            pl.BlockSpec((hidden, 2 * C), lambda b: (0, 0)),
            pl.BlockSpec((hidden, C), lambda b: (0, 0)),
            pl.BlockSpec((hidden, C), lambda b: (0, 0)),
        ],
        out_specs=pl.BlockSpec((nb, H * W, C), lambda b: (b, 0, 0)),
        out_shape=jax.ShapeDtypeStruct((B, H * W, C), jnp.float32),
        compiler_params=pltpu.CompilerParams(
            dimension_semantics=("parallel",),
            vmem_limit_bytes=60 * 1024 * 1024),
    )(xr, yr, yr,
      jax.lax.transpose(w_fc.astype(jnp.float32), (1, 0)),
      w_fc1.astype(jnp.float32), w_fc2.astype(jnp.float32))

    return jax.lax.transpose(out.reshape(B, H, W, C), (0, 3, 1, 2))


# speedup vs baseline: 1.0918x; 1.0918x over previous
import numpy as np
import jax
import jax.numpy as jnp
from jax.experimental import pallas as pl
from jax.experimental.pallas import tpu as pltpu

_HI = jax.lax.Precision.HIGHEST
_NB = 2


def _make_body(nb, c, hh, ww):
    inv_hw = np.float32(1.0 / (hh * ww))

    def body(x_ref, ya_ref, yb_ref, wf_ref, w1_ref, w2_ref, o_ref):
        x = x_ref[...]

        def taps(y_ref):
            y7 = y_ref[...].reshape(nb, hh // 2, 2, ww, 2, c // 128, 128)
            return (y7[:, :, 0, :, 0] + y7[:, :, 0, :, 1] +
                    y7[:, :, 1, :, 0] + y7[:, :, 1, :, 1])

        yp = jnp.concatenate([taps(ya_ref), taps(yb_ref)], axis=1) \
            .reshape(nb, hh * ww, c)

        dn = (((2,), (1,)), ((0,), (0,)))
        ones = jnp.full((nb, 1, hh * ww), inv_hw, jnp.float32)
        ones4 = jnp.full((nb, 1, hh * ww), inv_hw * np.float32(0.25),
                         jnp.float32)
        xg = jax.lax.dot_general(ones, x, dn,
                                 preferred_element_type=jnp.float32)
        yg = jax.lax.dot_general(ones4, yp, dn,
                                 preferred_element_type=jnp.float32)
        feat = jnp.concatenate([xg, yg], axis=2)[:, 0, :]

        common = jnp.maximum(
            jax.lax.dot_general(feat, wf_ref[...],
                                (((1,), (1,)), ((), ())), precision=_HI,
                                preferred_element_type=jnp.float32),
            0.0)
        xw = jax.nn.sigmoid(
            jnp.dot(common, w1_ref[...], precision=_HI,
                    preferred_element_type=jnp.float32))
        yw = jax.nn.sigmoid(
            jnp.dot(common, w2_ref[...], precision=_HI,
                    preferred_element_type=jnp.float32)) \
            * np.float32(0.25)

        o_ref[...] = x * xw[:, None, :] + yw[:, None, :] * yp

    return body


@jax.jit
def kernel(x, y, w_fc, w_fc1, w_fc2):
    B, C, H, W = x.shape
    assert y.shape == (B, C, 2 * H, 2 * W)
    hidden = w_fc.shape[1]

    nb = _NB if B % _NB == 0 else 1
    xt = jax.lax.transpose(x.astype(jnp.float32), (0, 2, 3, 1))
    yt = jax.lax.transpose(y.astype(jnp.float32), (0, 2, 3, 1))
    xr = xt.reshape(B, H * W, C)
    yr = yt.reshape(B, 4 * H * W, C)

    out = pl.pallas_call(
        _make_body(nb, C, H, W),
        grid=(B // nb,),
        in_specs=[
            pl.BlockSpec((nb, H * W, C), lambda b: (b, 0, 0)),
            pl.BlockSpec((nb, 2 * H * W, C), lambda b: (b, 0, 0)),
            pl.BlockSpec((nb, 2 * H * W, C), lambda b: (b, 1, 0)),
            pl.BlockSpec((hidden, 2 * C), lambda b: (0, 0)),
            pl.BlockSpec((hidden, C), lambda b: (0, 0)),
            pl.BlockSpec((hidden, C), lambda b: (0, 0)),
        ],
        out_specs=pl.BlockSpec((nb, H * W, C), lambda b: (b, 0, 0)),
        out_shape=jax.ShapeDtypeStruct((B, H * W, C), jnp.float32),
        compiler_params=pltpu.CompilerParams(
            dimension_semantics=("parallel",),
            vmem_limit_bytes=56 * 1024 * 1024),
    )(xr, yr, yr,
      jax.lax.transpose(w_fc.astype(jnp.float32), (1, 0)),
      w_fc1.astype(jnp.float32), w_fc2.astype(jnp.float32))

    return jax.lax.transpose(out.reshape(B, H, W, C), (0, 3, 1, 2))
